# Initial kernel scaffold; baseline (speedup 1.0000x reference)
import jax
import jax.numpy as jnp
from jax.experimental import pallas as pl


def kernel(s0, s1, Wpre, bpre, gpre, bepre, Wg, bg, gg, beg, edge_index, in_degree, out_degree, mat):
    raise NotImplementedError("write your pallas kernel here")


# speedup vs baseline: 6.2784x; 6.2784x over previous
import functools

import jax
import jax.numpy as jnp
from jax import lax
from jax.experimental import pallas as pl
from jax.experimental.pallas import tpu as pltpu
from jax.experimental.pallas import tpu_sc as plsc

N = 10000
E = 320000
D = 128
EPS = 1e-5

NC = 2
NS = 16
NP = 10240
EP = 327680
EPW = EP // (NC * NS)
CH = 128
NCHUNK = EPW // CH
RPT = NP // NS

_mesh = plsc.VectorSubcoreMesh(core_axis_name="c", subcore_axis_name="s")


def _bn(x, gamma, beta):
    mu = jnp.mean(x, axis=0)
    var = jnp.mean((x - mu) ** 2, axis=0)
    return (x - mu) * lax.rsqrt(var + EPS) * gamma + beta


@functools.partial(
    pl.kernel,
    out_type=jax.ShapeDtypeStruct((NC, NP, 16), jnp.float32),
    mesh=_mesh,
    scratch_types=[
        pltpu.VMEM((CH,), jnp.int32),
        pltpu.VMEM((CH, 16), jnp.float32),
        pltpu.VMEM_SHARED((NP, 16), jnp.float32),
    ],
)
def _deg_kernel(dst_hbm, zeros_hbm, out_hbm, dst_v, ones_v, cnt_sh):
    cid = lax.axis_index("c")
    sid = lax.axis_index("s")

    def fill(i, c):
        ones_v[i] = jnp.ones((16,), jnp.float32)
        return c

    lax.fori_loop(0, CH, fill, 0)

    r0 = sid * RPT
    pltpu.sync_copy(zeros_hbm.at[pl.ds(r0, RPT), :], cnt_sh.at[pl.ds(r0, RPT), :])
    plsc.subcore_barrier()

    base = (cid * NS + sid) * EPW

    def chunk(j, c):
        off = base + j * CH
        pltpu.sync_copy(dst_hbm.at[pl.ds(off, CH)], dst_v)
        pltpu.sync_copy(ones_v, cnt_sh.at[dst_v], add=True)
        return c

    lax.fori_loop(0, NCHUNK, chunk, 0)
    plsc.subcore_barrier()
    pltpu.sync_copy(cnt_sh.at[pl.ds(r0, RPT), :], out_hbm.at[cid, pl.ds(r0, RPT), :])


@functools.partial(
    pl.kernel,
    out_type=jax.ShapeDtypeStruct((NC, NP, D), jnp.float32),
    mesh=_mesh,
    scratch_types=[
        pltpu.VMEM((CH,), jnp.int32),
        pltpu.VMEM((CH,), jnp.int32),
        pltpu.VMEM((CH, D), jnp.float32),
        pltpu.VMEM_SHARED((NP, D), jnp.float32),
        pltpu.SemaphoreType.DMA,
    ],
)
def _edge_kernel(g_hbm, src_hbm, dst_hbm, zeros_hbm, out_hbm,
                 src_v, dst_v, rows_v, agg_sh, sem):
    cid = lax.axis_index("c")
    sid = lax.axis_index("s")

    r0 = sid * RPT
    pltpu.sync_copy(zeros_hbm.at[pl.ds(r0, RPT), :], agg_sh.at[pl.ds(r0, RPT), :])
    plsc.subcore_barrier()

    base = (cid * NS + sid) * EPW

    def chunk(j, c):
        off = base + j * CH
        pltpu.sync_copy(src_hbm.at[pl.ds(off, CH)], src_v)
        pltpu.sync_copy(dst_hbm.at[pl.ds(off, CH)], dst_v)
        pltpu.async_copy(g_hbm.at[src_v], rows_v, sem).wait()
        pltpu.sync_copy(rows_v, agg_sh.at[dst_v], add=True)
        return c

    lax.fori_loop(0, NCHUNK, chunk, 0)
    plsc.subcore_barrier()
    pltpu.sync_copy(agg_sh.at[pl.ds(r0, RPT), :], out_hbm.at[cid, pl.ds(r0, RPT), :])


def _pre_body(s0, s1, Wpre, bpre, gpre, bepre, Wg, cnt, ga_o, gb_o, dinv_o):
    deg = jnp.sum(cnt[0, :N, :] + cnt[1, :N, :], axis=1) + 1.0
    dinv = lax.rsqrt(deg)[:, None]
    dinv_o[...] = dinv

    n0 = _bn(jnp.dot(s0[...], Wpre[0], preferred_element_type=jnp.float32) + bpre[0],
             gpre[0], bepre[0])
    ga_o[:N, :] = jnp.dot(jax.nn.relu(n0), Wg[0], preferred_element_type=jnp.float32) * dinv
    ga_o[N:, :] = jnp.zeros((NP - N, D), jnp.float32)

    n1 = _bn(jnp.dot(s1[...], Wpre[1], preferred_element_type=jnp.float32) + bpre[1],
             gpre[1], bepre[1])
    gb_o[:N, :] = jnp.dot(jax.nn.relu(n1), Wg[1], preferred_element_type=jnp.float32) * dinv
    gb_o[N:, :] = jnp.zeros((NP - N, D), jnp.float32)


def _mid_body(pa, pb, ga, gb, dinv, bg, gg, beg, Wg, x2_o, g2_o):
    dv = dinv[...]
    tu = _bn(dv * (pa[0, :N, :] + pa[1, :N, :] + ga[:N, :]) + bg[0], gg[0], beg[0])
    tv = _bn(dv * (pb[0, :N, :] + pb[1, :N, :] + gb[:N, :]) + bg[1], gg[1], beg[1])
    x2 = tu + tv
    x2_o[...] = x2
    g2_o[:N, :] = jnp.dot(jax.nn.relu(x2), Wg[2], preferred_element_type=jnp.float32) * dv
    g2_o[N:, :] = jnp.zeros((NP - N, D), jnp.float32)


def _post_body(p2, g2, x2, dinv, bg, gg, beg, out_o):
    x3 = _bn(dinv[...] * (p2[0, :N, :] + p2[1, :N, :] + g2[:N, :]) + bg[2],
             gg[2], beg[2])
    out_o[:, :D] = x2[...]
    out_o[:, D:] = x3


_f32 = jnp.float32

_tc_params = pltpu.CompilerParams(vmem_limit_bytes=100 * 1024 * 1024)

_pre_call = pl.pallas_call(
    _pre_body,
    compiler_params=_tc_params,
    out_shape=(
        jax.ShapeDtypeStruct((NP, D), _f32),
        jax.ShapeDtypeStruct((NP, D), _f32),
        jax.ShapeDtypeStruct((N, 1), _f32),
    ),
)

_mid_call = pl.pallas_call(
    _mid_body,
    compiler_params=_tc_params,
    out_shape=(
        jax.ShapeDtypeStruct((N, D), _f32),
        jax.ShapeDtypeStruct((NP, D), _f32),
    ),
)

_post_call = pl.pallas_call(
    _post_body,
    compiler_params=_tc_params,
    out_shape=jax.ShapeDtypeStruct((N, 2 * D), _f32),
)


def kernel(s0, s1, Wpre, bpre, gpre, bepre, Wg, bg, gg, beg, edge_index,
           in_degree, out_degree, mat):
    pad = jnp.full((EP - E,), N, jnp.int32)
    src = jnp.concatenate([edge_index[0], pad])
    dst = jnp.concatenate([edge_index[1], pad])
    zeros16 = jnp.zeros((NP, 16), _f32)
    zerosD = jnp.zeros((NP, D), _f32)

    cnt = _deg_kernel(dst, zeros16)
    ga, gb, dinv = _pre_call(s0, s1, Wpre, bpre, gpre, bepre, Wg, cnt)
    pa = _edge_kernel(ga, src, dst, zerosD)
    pb = _edge_kernel(gb, src, dst, zerosD)
    x2, g2 = _mid_call(pa, pb, ga, gb, dinv, bg, gg, beg, Wg)
    p2 = _edge_kernel(g2, src, dst, zerosD)
    return _post_call(p2, g2, x2, dinv, bg, gg, beg)


# speedup vs baseline: 13.5342x; 2.1557x over previous
import functools

import jax
import jax.numpy as jnp
from jax import lax
from jax.experimental import pallas as pl
from jax.experimental.pallas import tpu as pltpu
from jax.experimental.pallas import tpu_sc as plsc

N = 10000
E = 320000
D = 128
EPS = 1e-5

NC = 2
NS = 16
NP = 10240
EP = 327680
EPW = EP // (NC * NS)
CH = 128
NCHUNK = EPW // CH
RPT = NP // NS

_mesh = plsc.VectorSubcoreMesh(core_axis_name="c", subcore_axis_name="s")


def _bn(x, gamma, beta):
    mu = jnp.mean(x, axis=0)
    var = jnp.mean((x - mu) ** 2, axis=0)
    return (x - mu) * lax.rsqrt(var + EPS) * gamma + beta


@functools.partial(
    pl.kernel,
    out_type=jax.ShapeDtypeStruct((NC, NP, 16), jnp.float32),
    mesh=_mesh,
    scratch_types=[
        pltpu.VMEM((CH,), jnp.int32),
        pltpu.VMEM((CH, 16), jnp.float32),
        pltpu.VMEM_SHARED((NP, 16), jnp.float32),
    ],
)
def _deg_kernel(dst_hbm, zeros_hbm, out_hbm, dst_v, ones_v, cnt_sh):
    cid = lax.axis_index("c")
    sid = lax.axis_index("s")

    def fill(i, c):
        ones_v[i] = jnp.ones((16,), jnp.float32)
        return c

    lax.fori_loop(0, CH, fill, 0)

    r0 = sid * RPT
    pltpu.sync_copy(zeros_hbm.at[pl.ds(r0, RPT), :], cnt_sh.at[pl.ds(r0, RPT), :])
    plsc.subcore_barrier()

    base = (cid * NS + sid) * EPW

    def chunk(j, c):
        off = base + j * CH
        pltpu.sync_copy(dst_hbm.at[pl.ds(off, CH)], dst_v)
        pltpu.sync_copy(ones_v, cnt_sh.at[dst_v], add=True)
        return c

    lax.fori_loop(0, NCHUNK, chunk, 0)
    plsc.subcore_barrier()
    pltpu.sync_copy(cnt_sh.at[pl.ds(r0, RPT), :], out_hbm.at[cid, pl.ds(r0, RPT), :])


@functools.partial(
    pl.kernel,
    out_type=jax.ShapeDtypeStruct((NC, NP, D), jnp.float32),
    mesh=_mesh,
    scratch_types=[
        pltpu.VMEM((CH,), jnp.int32),
        pltpu.VMEM((CH,), jnp.int32),
        pltpu.VMEM((CH, D), jnp.float32),
        pltpu.VMEM_SHARED((NP, D), jnp.float32),
        pltpu.SemaphoreType.DMA,
    ],
)
def _edge_kernel(g_hbm, src_hbm, dst_hbm, zeros_hbm, out_hbm,
                 src_v, dst_v, rows_v, agg_sh, sem):
    cid = lax.axis_index("c")
    sid = lax.axis_index("s")

    r0 = sid * RPT
    pltpu.sync_copy(zeros_hbm.at[pl.ds(r0, RPT), :], agg_sh.at[pl.ds(r0, RPT), :])
    plsc.subcore_barrier()

    base = (cid * NS + sid) * EPW

    def chunk(j, c):
        off = base + j * CH
        pltpu.sync_copy(src_hbm.at[pl.ds(off, CH)], src_v)
        pltpu.sync_copy(dst_hbm.at[pl.ds(off, CH)], dst_v)
        pltpu.async_copy(g_hbm.at[src_v], rows_v, sem).wait()
        pltpu.sync_copy(rows_v, agg_sh.at[dst_v], add=True)
        return c

    lax.fori_loop(0, NCHUNK, chunk, 0)
    plsc.subcore_barrier()
    pltpu.sync_copy(agg_sh.at[pl.ds(r0, RPT), :], out_hbm.at[cid, pl.ds(r0, RPT), :])


def _pre_body(s0, s1, Wpre, bpre, gpre, bepre, Wg, cnt, ga_o, gb_o, dinv_o):
    deg = jnp.sum(cnt[0, :N, :] + cnt[1, :N, :], axis=1) + 1.0
    dinv = lax.rsqrt(deg)[:, None]
    dinv_o[...] = dinv

    n0 = _bn(jnp.dot(s0[...], Wpre[0], preferred_element_type=jnp.float32) + bpre[0],
             gpre[0], bepre[0])
    ga_o[:N, :] = jnp.dot(jax.nn.relu(n0), Wg[0], preferred_element_type=jnp.float32) * dinv
    ga_o[N:, :] = jnp.zeros((NP - N, D), jnp.float32)

    n1 = _bn(jnp.dot(s1[...], Wpre[1], preferred_element_type=jnp.float32) + bpre[1],
             gpre[1], bepre[1])
    gb_o[:N, :] = jnp.dot(jax.nn.relu(n1), Wg[1], preferred_element_type=jnp.float32) * dinv
    gb_o[N:, :] = jnp.zeros((NP - N, D), jnp.float32)


def _mid_body(pa, pb, ga, gb, dinv, bg, gg, beg, Wg, x2_o, g2_o):
    dv = dinv[...]
    tu = _bn(dv * (pa[0, :N, :] + pa[1, :N, :] + ga[:N, :]) + bg[0], gg[0], beg[0])
    tv = _bn(dv * (pb[0, :N, :] + pb[1, :N, :] + gb[:N, :]) + bg[1], gg[1], beg[1])
    x2 = tu + tv
    x2_o[...] = x2
    g2_o[:N, :] = jnp.dot(jax.nn.relu(x2), Wg[2], preferred_element_type=jnp.float32) * dv
    g2_o[N:, :] = jnp.zeros((NP - N, D), jnp.float32)


def _post_body(p2, g2, x2, dinv, bg, gg, beg, out_o):
    x3 = _bn(dinv[...] * (p2[0, :N, :] + p2[1, :N, :] + g2[:N, :]) + bg[2],
             gg[2], beg[2])
    out_o[:, :D] = x2[...]
    out_o[:, D:] = x3


_f32 = jnp.float32

_tc_params = pltpu.CompilerParams(vmem_limit_bytes=100 * 1024 * 1024)

_pre_call = pl.pallas_call(
    _pre_body,
    compiler_params=_tc_params,
    out_shape=(
        jax.ShapeDtypeStruct((NP, D), _f32),
        jax.ShapeDtypeStruct((NP, D), _f32),
        jax.ShapeDtypeStruct((N, 1), _f32),
    ),
)

_mid_call = pl.pallas_call(
    _mid_body,
    compiler_params=_tc_params,
    out_shape=(
        jax.ShapeDtypeStruct((N, D), _f32),
        jax.ShapeDtypeStruct((NP, D), _f32),
    ),
)

_post_call = pl.pallas_call(
    _post_body,
    compiler_params=_tc_params,
    out_shape=jax.ShapeDtypeStruct((N, 2 * D), _f32),
)


def kernel(s0, s1, Wpre, bpre, gpre, bepre, Wg, bg, gg, beg, edge_index,
           in_degree, out_degree, mat):
    pad = N + jnp.arange(EP - E, dtype=jnp.int32) % (NP - N)
    src = jnp.concatenate([edge_index[0], pad])
    dst = jnp.concatenate([edge_index[1], pad])
    zeros16 = jnp.zeros((NP, 16), _f32)
    zerosD = jnp.zeros((NP, D), _f32)

    cnt = _deg_kernel(dst, zeros16)
    ga, gb, dinv = _pre_call(s0, s1, Wpre, bpre, gpre, bepre, Wg, cnt)
    pa = _edge_kernel(ga, src, dst, zerosD)
    pb = _edge_kernel(gb, src, dst, zerosD)
    x2, g2 = _mid_call(pa, pb, ga, gb, dinv, bg, gg, beg, Wg)
    p2 = _edge_kernel(g2, src, dst, zerosD)
    return _post_call(p2, g2, x2, dinv, bg, gg, beg)
